# baseline (speedup 1.0000x reference)
import functools

import jax
import jax.numpy as jnp
from jax import lax
from jax.experimental import pallas as pl
from jax.experimental.pallas import tpu as pltpu
from jax.experimental.pallas import tpu_sc as plsc

N = 16384
NSUB = 16
SLICE = N // NSUB
NVREG = SLICE // 16
NR = 32
CELL = 96
RCAP = 1024
KPAD = RCAP + 16
IMAX = 0x7FFFFFFF


def _iota16():
    return lax.iota(jnp.int32, 16)


def _sc_body(pred_hbm, grp_hbm, out_hbm,
             vbuf, gbuf, hist64, cellcnt, staging, histall, recvbuf,
             keybuf, sumv, sumall, hist_sh, cells_sh, sums_sh):
    cid = lax.axis_index("c")
    sid = lax.axis_index("s")
    r_own = cid * NSUB + sid
    iota = _iota16()

    pltpu.sync_copy(pred_hbm.at[pl.ds(sid * SLICE, SLICE)], vbuf)
    pltpu.sync_copy(grp_hbm.at[pl.ds(sid * SLICE, SLICE)], gbuf)

    zeros = jnp.zeros((16,), jnp.int32)
    for k in range(4):
        hist64[pl.ds(16 * k, 16)] = zeros
    cellcnt[...] = zeros

    def scan_body(i, _):
        v = vbuf[pl.ds(i * 16, 16)]
        g = gbuf[pl.ds(i * 16, 16)]
        d = jnp.minimum((v * jnp.float32(NR)).astype(jnp.int32), NR - 1)
        p = lax.shift_left(plsc.bitcast(v, jnp.int32), 1) | g
        dg = d * 2 + g
        occ_dg, last_dg = plsc.scan_count(dg)
        cur = plsc.load_gather(hist64, [dg])
        plsc.store_scatter(hist64, [dg], cur + occ_dg, mask=last_dg)

        own = (d >= cid * NSUB) & (d < cid * NSUB + NSUB)
        occ_d, last_d = plsc.scan_count(d, mask=own)
        dl = d & (NSUB - 1)
        base = plsc.load_gather(cellcnt, [dl])
        slot = jnp.minimum(base + occ_d - 1, CELL - 1)
        plsc.store_scatter(staging, [dl * CELL + slot], p, mask=own)
        plsc.store_scatter(cellcnt, [dl], slot + 1, mask=last_d & own)
        return 0

    lax.fori_loop(0, NVREG, scan_body, 0)

    pltpu.sync_copy(hist64, hist_sh.at[pl.ds(sid * 64, 64)])
    pltpu.sync_copy(staging, cells_sh.at[pl.ds(sid * (NSUB * CELL),
                                               NSUB * CELL)])
    plsc.subcore_barrier()

    pltpu.sync_copy(hist_sh, histall)
    b0 = 2 * r_own
    acc = [jnp.zeros((16,), jnp.int32) for _ in range(4)]
    cnt_t = []
    for t in range(NSUB):
        c_t = jnp.int32(0)
        for k in range(4):
            hv = histall[pl.ds(t * 64 + 16 * k, 16)]
            acc[k] = acc[k] + hv
            binv = iota + 16 * k
            sel = (binv == b0) | (binv == b0 + 1)
            c_t = c_t + jnp.sum(jnp.where(sel, hv, 0))
        cnt_t.append(c_t)
    P0 = jnp.int32(0)
    P1 = jnp.int32(0)
    n0 = jnp.int32(0)
    for k in range(4):
        binv = iota + 16 * k
        rv = lax.shift_right_logical(binv, 1)
        is0 = (binv & 1) == 0
        below = rv < r_own
        P0 = P0 + jnp.sum(jnp.where(is0 & below, acc[k], 0))
        P1 = P1 + jnp.sum(jnp.where((~is0) & below, acc[k], 0))
        n0 = n0 + jnp.sum(jnp.where(is0, acc[k], 0))
    m = jnp.minimum(n0, N - n0)

    for t in range(NSUB):
        pltpu.sync_copy(
            cells_sh.at[pl.ds(t * (NSUB * CELL) + sid * CELL, CELL)],
            recvbuf.at[pl.ds(t * CELL, CELL)])

    maxv = jnp.full((16,), IMAX, jnp.int32)

    def pad_body(i, _):
        keybuf[pl.ds(i * 16, 16)] = maxv
        return 0

    lax.fori_loop(0, KPAD // 16, pad_body, 0)

    off = jnp.int32(0)
    for t in range(NSUB):
        for j in range(CELL // 16):
            x = recvbuf[pl.ds((t * (CELL // 16) + j) * 16, 16)]
            msk = (iota + 16 * j) < cnt_t[t]
            plsc.store_compressed(keybuf.at[pl.ds(off, 16)], x, mask=msk)
            off = jnp.minimum(off + jnp.sum(msk.astype(jnp.int32)),
                              jnp.int32(RCAP))
    c_r = off

    def vsort_body(i, _):
        keybuf[pl.ds(i * 16, 16)] = jnp.sort(keybuf[pl.ds(i * 16, 16)])
        return 0

    lax.fori_loop(0, RCAP // 16, vsort_body, 0, unroll=4)

    nv = RCAP // 16
    for lg in range(1, 7):
        mm = 1 << (lg - 1)
        if mm == 1:
            def split1(z, _):
                a = keybuf[pl.ds(z * 32, 16)]
                b = keybuf[pl.ds(z * 32 + 16, 16)]
                rb = lax.rev(b, (0,))
                keybuf[pl.ds(z * 32, 16)] = jnp.minimum(a, rb)
                keybuf[pl.ds(z * 32 + 16, 16)] = jnp.maximum(a, rb)
                return 0

            lax.fori_loop(0, nv // 2, split1, 0, unroll=4)
        else:
            def splitm(z, _, lg=lg, mm=mm):
                q = lax.shift_right_logical(z, lg - 2)
                i = z & (mm // 2 - 1)
                basea = 2 * q * mm
                a1 = keybuf[pl.ds((basea + i) * 16, 16)]
                a2 = keybuf[pl.ds((basea + mm - 1 - i) * 16, 16)]
                b1 = keybuf[pl.ds((basea + mm + i) * 16, 16)]
                b2 = keybuf[pl.ds((basea + 2 * mm - 1 - i) * 16, 16)]
                rb1 = lax.rev(b1, (0,))
                rb2 = lax.rev(b2, (0,))
                keybuf[pl.ds((basea + i) * 16, 16)] = jnp.minimum(a1, rb2)
                keybuf[pl.ds((basea + mm + i) * 16, 16)] = jnp.maximum(a1, rb2)
                keybuf[pl.ds((basea + mm - 1 - i) * 16, 16)] = (
                    jnp.minimum(a2, rb1))
                keybuf[pl.ds((basea + 2 * mm - 1 - i) * 16, 16)] = (
                    jnp.maximum(a2, rb1))
                return 0

            lax.fori_loop(0, nv // 4, splitm, 0, unroll=2)
        for s in range(lg - 2, -1, -1):
            def sub_body(w, _, s=s):
                v = lax.shift_left(lax.shift_right_logical(w, s), s + 1) | (
                    w & ((1 << s) - 1))
                u = v + (1 << s)
                x = keybuf[pl.ds(v * 16, 16)]
                y = keybuf[pl.ds(u * 16, 16)]
                keybuf[pl.ds(v * 16, 16)] = jnp.minimum(x, y)
                keybuf[pl.ds(u * 16, 16)] = jnp.maximum(x, y)
                return 0

            lax.fori_loop(0, nv // 2, sub_body, 0, unroll=4)
        lax.fori_loop(0, nv, vsort_body, 0, unroll=4)

    def walk_body(k, carry):
        c1run, accv = carry
        q = keybuf[pl.ds(k * 16, 16)]
        pos = iota + k * 16
        validm = pos < c_r
        f = jnp.where(validm, q & 1, 0)
        incl = plsc.cumsum(f)
        l1 = incl - f + c1run
        l0 = pos - l1
        i0g = P0 + l0
        i1g = P1 + l1
        isg1 = (q & 1) == 1
        iown = jnp.where(isg1, i1g, i0g)
        roth = jnp.where(isg1, i0g, i1g)
        cond = validm & (iown < m)
        val = plsc.bitcast(lax.shift_right_logical(q, 1), jnp.float32)
        sgn = jnp.where(roth > iown, jnp.float32(1.0), jnp.float32(-1.0))
        accv = accv + jnp.where(cond, sgn * val, jnp.float32(0.0))
        return c1run + jnp.sum(f), accv

    _, accv = lax.fori_loop(0, RCAP // 16, walk_body,
                            (jnp.int32(0), jnp.zeros((16,), jnp.float32)),
                            unroll=2)

    sumv[...] = accv
    pltpu.sync_copy(sumv, sums_sh.at[pl.ds(sid * 16, 16)])
    plsc.subcore_barrier()

    @pl.when(sid == 0)
    def _():
        pltpu.sync_copy(sums_sh, sumall)
        tot = jnp.zeros((16,), jnp.float32)
        for t in range(NSUB):
            tot = tot + sumall[pl.ds(t * 16, 16)]
        stot = jnp.sum(tot)
        outv = jnp.where(iota == 0, stot,
                         jnp.where(iota == 1, m.astype(jnp.float32),
                                   jnp.float32(0.0)))
        sumv[...] = outv
        pltpu.sync_copy(sumv, out_hbm.at[pl.ds(cid * 16, 16)])


@functools.partial(jax.jit, static_argnames=())
def _sc_call(pred, grp):
    f = pl.kernel(
        _sc_body,
        out_type=jax.ShapeDtypeStruct((32,), jnp.float32),
        mesh=plsc.VectorSubcoreMesh(core_axis_name="c",
                                    subcore_axis_name="s",
                                    num_cores=2, num_subcores=NSUB),
        scratch_types=[
            pltpu.VMEM((SLICE,), jnp.float32),
            pltpu.VMEM((SLICE,), jnp.int32),
            pltpu.VMEM((64,), jnp.int32),
            pltpu.VMEM((16,), jnp.int32),
            pltpu.VMEM((NSUB * CELL,), jnp.int32),
            pltpu.VMEM((NSUB * 64,), jnp.int32),
            pltpu.VMEM((NSUB * CELL,), jnp.int32),
            pltpu.VMEM((KPAD,), jnp.int32),
            pltpu.VMEM((16,), jnp.float32),
            pltpu.VMEM((NSUB * 16,), jnp.float32),
            pltpu.VMEM_SHARED((NSUB * 64,), jnp.int32),
            pltpu.VMEM_SHARED((NSUB * NSUB * CELL,), jnp.int32),
            pltpu.VMEM_SHARED((NSUB * 16,), jnp.float32),
        ],
        compiler_params=pltpu.CompilerParams(needs_layout_passes=False),
    )
    return f(pred, grp)


def kernel(batch_pred, batch_group):
    out = _sc_call(batch_pred, batch_group.astype(jnp.int32))
    return (out[0] + out[16]) / out[1]


# speedup vs baseline: 1.1025x; 1.1025x over previous
import functools

import jax
import jax.numpy as jnp
from jax import lax
from jax.experimental import pallas as pl
from jax.experimental.pallas import tpu as pltpu
from jax.experimental.pallas import tpu_sc as plsc

N = 16384
NSUB = 16
SLICE = N // NSUB
NVREG = SLICE // 16
NR = 32
CELL = 48
RBLK = 2 * CELL
RCAP = 1024
KPAD = RCAP + 16
IMAX = 0x7FFFFFFF


def _sort16(x):
    return jnp.sort(x)


def _merge_unit(vs):
    un = len(vs)
    m = un // 2
    rb = [lax.rev(vs[un - 1 - j], (0,)) for j in range(m)]
    lst = ([jnp.minimum(vs[j], rb[j]) for j in range(m)] +
           [jnp.maximum(vs[j], rb[j]) for j in range(m)])
    d = m // 2
    while d >= 1:
        nxt = list(lst)
        for v in range(un):
            if (v % (2 * d)) < d:
                nxt[v] = jnp.minimum(lst[v], lst[v + d])
                nxt[v + d] = jnp.maximum(lst[v], lst[v + d])
        lst = nxt
        d //= 2
    return [_sort16(x) for x in lst]


def _sc_body(pred_hbm, grp_hbm, out_hbm,
             vbuf, gbuf, hist64, staging, histall, recvbuf,
             keybuf, sumv, sumall, hist_sh, cells_sh, sums_sh, dsem):
    cid = lax.axis_index("c")
    sid = lax.axis_index("s")
    r_own = cid * NSUB + sid
    iota = lax.iota(jnp.int32, 16)

    cp1 = pltpu.async_copy(pred_hbm.at[pl.ds(sid * SLICE, SLICE)], vbuf, dsem)
    cp2 = pltpu.async_copy(grp_hbm.at[pl.ds(sid * SLICE, SLICE)], gbuf, dsem)
    cp1.wait()
    cp2.wait()

    zeros = jnp.zeros((16,), jnp.int32)
    for k in range(4):
        hist64[pl.ds(16 * k, 16)] = zeros

    base_bin = cid * NR

    def scan_body(i, _):
        v = vbuf[pl.ds(i * 16, 16)]
        g = gbuf[pl.ds(i * 16, 16)]
        d = jnp.minimum((v * jnp.float32(NR)).astype(jnp.int32), NR - 1)
        p = lax.shift_left(plsc.bitcast(v, jnp.int32), 1) | g
        dg = d * 2 + g
        occ, lastm = plsc.scan_count(dg)
        cur = plsc.load_gather(hist64, [dg])
        plsc.store_scatter(hist64, [dg], cur + occ, mask=lastm)
        own = (dg >= base_bin) & (dg < base_bin + NR)
        slot = jnp.minimum(cur + occ - 1, CELL - 1)
        plsc.store_scatter(staging, [(dg & (NR - 1)) * CELL + slot], p,
                           mask=own)
        return 0

    lax.fori_loop(0, NVREG, scan_body, 0, unroll=2)

    cps = [pltpu.async_copy(hist64, hist_sh.at[pl.ds(sid * 64, 64)], dsem)]
    for r in range(NSUB):
        cps.append(pltpu.async_copy(
            staging.at[pl.ds(r * RBLK, RBLK)],
            cells_sh.at[pl.ds(r * (NSUB * RBLK) + sid * RBLK, RBLK)], dsem))
    for c in cps:
        c.wait()
    plsc.subcore_barrier()

    cprecv = pltpu.async_copy(
        cells_sh.at[pl.ds(sid * (NSUB * RBLK), NSUB * RBLK)], recvbuf, dsem)
    pltpu.sync_copy(hist_sh, histall)
    b0 = 2 * r_own
    acc = [jnp.zeros((16,), jnp.int32) for _ in range(4)]
    cnt0 = []
    cnt1 = []
    for t in range(NSUB):
        c0 = jnp.int32(0)
        c1 = jnp.int32(0)
        for k in range(4):
            hv = histall[pl.ds(t * 64 + 16 * k, 16)]
            acc[k] = acc[k] + hv
            binv = iota + 16 * k
            c0 = c0 + jnp.sum(jnp.where(binv == b0, hv, 0))
            c1 = c1 + jnp.sum(jnp.where(binv == b0 + 1, hv, 0))
        cnt0.append(jnp.minimum(c0, CELL))
        cnt1.append(jnp.minimum(c1, CELL))
    P0 = jnp.int32(0)
    P1 = jnp.int32(0)
    n0 = jnp.int32(0)
    for k in range(4):
        binv = iota + 16 * k
        rv = lax.shift_right_logical(binv, 1)
        is0 = (binv & 1) == 0
        below = rv < r_own
        P0 = P0 + jnp.sum(jnp.where(is0 & below, acc[k], 0))
        P1 = P1 + jnp.sum(jnp.where((~is0) & below, acc[k], 0))
        n0 = n0 + jnp.sum(jnp.where(is0, acc[k], 0))
    m = jnp.minimum(n0, N - n0)

    maxv = jnp.full((16,), IMAX, jnp.int32)

    def pad_body(i, _):
        keybuf[pl.ds(i * 16, 16)] = maxv
        return 0

    lax.fori_loop(0, KPAD // 16, pad_body, 0, unroll=4)
    cprecv.wait()

    off = jnp.int32(0)
    for t in range(NSUB):
        for half, cnt in ((0, cnt0[t]), (1, cnt1[t])):
            cbase = t * RBLK + half * CELL
            for j in range(CELL // 16):
                x = recvbuf[pl.ds(cbase + j * 16, 16)]
                msk = (iota + 16 * j) < cnt
                dst = jnp.minimum(off + jnp.minimum(jnp.int32(16 * j), cnt),
                                  jnp.int32(RCAP))
                plsc.store_compressed(keybuf.at[pl.ds(dst, 16)], x, mask=msk)
            off = jnp.minimum(off + cnt, jnp.int32(RCAP))
    c_r = off

    for base in range(0, 64, 2):
        a = _sort16(keybuf[pl.ds(base * 16, 16)])
        b = _sort16(keybuf[pl.ds((base + 1) * 16, 16)])
        res = _merge_unit([a, b])
        keybuf[pl.ds(base * 16, 16)] = res[0]
        keybuf[pl.ds((base + 1) * 16, 16)] = res[1]
    for lg in range(2, 5):
        un = 1 << lg
        for base in range(0, 64, un):
            vs = [keybuf[pl.ds((base + j) * 16, 16)] for j in range(un)]
            res = _merge_unit(vs)
            for j in range(un):
                keybuf[pl.ds((base + j) * 16, 16)] = res[j]

    def _halfpass(base, m):
        for i in range(m // 2):
            a1 = keybuf[pl.ds((base + i) * 16, 16)]
            a2 = keybuf[pl.ds((base + m - 1 - i) * 16, 16)]
            b1 = keybuf[pl.ds((base + m + i) * 16, 16)]
            b2 = keybuf[pl.ds((base + 2 * m - 1 - i) * 16, 16)]
            rb1 = lax.rev(b1, (0,))
            rb2 = lax.rev(b2, (0,))
            keybuf[pl.ds((base + i) * 16, 16)] = jnp.minimum(a1, rb2)
            keybuf[pl.ds((base + m + i) * 16, 16)] = jnp.maximum(a1, rb2)
            keybuf[pl.ds((base + m - 1 - i) * 16, 16)] = jnp.minimum(a2, rb1)
            keybuf[pl.ds((base + 2 * m - 1 - i) * 16, 16)] = (
                jnp.maximum(a2, rb1))

    def _distpass(base, nvr, d):
        for w in range(nvr):
            if (w % (2 * d)) < d:
                x = keybuf[pl.ds((base + w) * 16, 16)]
                y = keybuf[pl.ds((base + w + d) * 16, 16)]
                keybuf[pl.ds((base + w) * 16, 16)] = jnp.minimum(x, y)
                keybuf[pl.ds((base + w + d) * 16, 16)] = jnp.maximum(x, y)

    def _finish16(base):
        lst = [keybuf[pl.ds((base + j) * 16, 16)] for j in range(16)]
        d = 8
        while d >= 1:
            nxt = list(lst)
            for v in range(16):
                if (v % (2 * d)) < d:
                    nxt[v] = jnp.minimum(lst[v], lst[v + d])
                    nxt[v + d] = jnp.maximum(lst[v], lst[v + d])
            lst = nxt
            d //= 2
        for j in range(16):
            keybuf[pl.ds((base + j) * 16, 16)] = _sort16(lst[j])

    for base in (0, 32):
        _halfpass(base, 16)
        _finish16(base)
        _finish16(base + 16)
    _halfpass(0, 32)
    _distpass(0, 64, 16)
    for base in (0, 16, 32, 48):
        _finish16(base)

    def walk_body(k, carry):
        c1run, accv = carry
        q = keybuf[pl.ds(k * 16, 16)]
        pos = iota + k * 16
        validm = pos < c_r
        f = jnp.where(validm, q & 1, 0)
        incl = plsc.cumsum(f)
        l1 = incl - f + c1run
        l0 = pos - l1
        i0g = P0 + l0
        i1g = P1 + l1
        isg1 = (q & 1) == 1
        iown = jnp.where(isg1, i1g, i0g)
        roth = jnp.where(isg1, i0g, i1g)
        cond = validm & (iown < m)
        val = plsc.bitcast(lax.shift_right_logical(q, 1), jnp.float32)
        sgn = jnp.where(roth > iown, jnp.float32(1.0), jnp.float32(-1.0))
        accv = accv + jnp.where(cond, sgn * val, jnp.float32(0.0))
        return c1run + jnp.sum(f), accv

    _, accv = lax.fori_loop(0, RCAP // 16, walk_body,
                            (jnp.int32(0), jnp.zeros((16,), jnp.float32)),
                            unroll=2)

    sumv[...] = accv
    pltpu.sync_copy(sumv, sums_sh.at[pl.ds(sid * 16, 16)])
    plsc.subcore_barrier()

    @pl.when(sid == 0)
    def _():
        pltpu.sync_copy(sums_sh, sumall)
        tot = jnp.zeros((16,), jnp.float32)
        for t in range(NSUB):
            tot = tot + sumall[pl.ds(t * 16, 16)]
        stot = jnp.sum(tot)
        outv = jnp.where(iota == 0, stot,
                         jnp.where(iota == 1, m.astype(jnp.float32),
                                   jnp.float32(0.0)))
        sumv[...] = outv
        pltpu.sync_copy(sumv, out_hbm.at[pl.ds(cid * 16, 16)])


@functools.partial(jax.jit, static_argnames=())
def _sc_call(pred, grp):
    f = pl.kernel(
        _sc_body,
        out_type=jax.ShapeDtypeStruct((32,), jnp.float32),
        mesh=plsc.VectorSubcoreMesh(core_axis_name="c",
                                    subcore_axis_name="s",
                                    num_cores=2, num_subcores=NSUB),
        scratch_types=[
            pltpu.VMEM((SLICE,), jnp.float32),
            pltpu.VMEM((SLICE,), jnp.int32),
            pltpu.VMEM((64,), jnp.int32),
            pltpu.VMEM((NR * CELL,), jnp.int32),
            pltpu.VMEM((NSUB * 64,), jnp.int32),
            pltpu.VMEM((NSUB * RBLK,), jnp.int32),
            pltpu.VMEM((KPAD,), jnp.int32),
            pltpu.VMEM((16,), jnp.float32),
            pltpu.VMEM((NSUB * 16,), jnp.float32),
            pltpu.VMEM_SHARED((NSUB * 64,), jnp.int32),
            pltpu.VMEM_SHARED((NSUB * NSUB * RBLK,), jnp.int32),
            pltpu.VMEM_SHARED((NSUB * 16,), jnp.float32),
            pltpu.SemaphoreType.DMA,
        ],
        compiler_params=pltpu.CompilerParams(needs_layout_passes=False),
    )
    return f(pred, grp)


def kernel(batch_pred, batch_group):
    out = _sc_call(batch_pred, batch_group.astype(jnp.int32))
    return (out[0] + out[16]) / out[1]


# speedup vs baseline: 1.1345x; 1.0290x over previous
import functools

import jax
import jax.numpy as jnp
from jax import lax
from jax.experimental import pallas as pl
from jax.experimental.pallas import tpu as pltpu
from jax.experimental.pallas import tpu_sc as plsc

N = 16384
NSUB = 16
SLICE = N // NSUB
NVREG = SLICE // 16
NR = 32
CELL = 48
RBLK = 2 * CELL
RCAP = 1024
KPAD = RCAP + 16
IMAX = 0x7FFFFFFF


def _sort16(x):
    return jnp.sort(x)


def _merge_unit(vs):
    un = len(vs)
    m = un // 2
    rb = [lax.rev(vs[un - 1 - j], (0,)) for j in range(m)]
    lst = ([jnp.minimum(vs[j], rb[j]) for j in range(m)] +
           [jnp.maximum(vs[j], rb[j]) for j in range(m)])
    d = m // 2
    while d >= 1:
        nxt = list(lst)
        for v in range(un):
            if (v % (2 * d)) < d:
                nxt[v] = jnp.minimum(lst[v], lst[v + d])
                nxt[v + d] = jnp.maximum(lst[v], lst[v + d])
        lst = nxt
        d //= 2
    return [_sort16(x) for x in lst]


def _sc_body(pred_hbm, grp_hbm, out_hbm,
             vbuf, gbuf, hist64, staging, histall, recvbuf,
             keybuf, sumv, sumall, hist_sh, cells_sh, sums_sh, dsem):
    cid = lax.axis_index("c")
    sid = lax.axis_index("s")
    r_own = cid * NSUB + sid
    iota = lax.iota(jnp.int32, 16)

    cp1 = pltpu.async_copy(pred_hbm.at[pl.ds(sid * SLICE, SLICE)], vbuf, dsem)
    cp2 = pltpu.async_copy(grp_hbm.at[pl.ds(sid * SLICE, SLICE)], gbuf, dsem)
    cp1.wait()
    cp2.wait()

    zeros = jnp.zeros((16,), jnp.int32)
    for k in range(4):
        hist64[pl.ds(16 * k, 16)] = zeros

    base_bin = cid * NR

    def scan_body(i, _):
        v = vbuf[pl.ds(i * 16, 16)]
        g = gbuf[pl.ds(i * 16, 16)]
        d = jnp.minimum((v * jnp.float32(NR)).astype(jnp.int32), NR - 1)
        p = lax.shift_left(plsc.bitcast(v, jnp.int32), 1) | g
        dg = d * 2 + g
        occ, lastm = plsc.scan_count(dg)
        cur = plsc.load_gather(hist64, [dg])
        plsc.store_scatter(hist64, [dg], cur + occ, mask=lastm)
        own = (dg >= base_bin) & (dg < base_bin + NR)
        slot = jnp.minimum(cur + occ - 1, CELL - 1)
        plsc.store_scatter(staging, [(dg & (NR - 1)) * CELL + slot], p,
                           mask=own)
        return 0

    lax.fori_loop(0, NVREG, scan_body, 0, unroll=2)

    cps = [pltpu.async_copy(hist64, hist_sh.at[pl.ds(sid * 64, 64)], dsem)]
    for r in range(NSUB):
        cps.append(pltpu.async_copy(
            staging.at[pl.ds(r * RBLK, RBLK)],
            cells_sh.at[pl.ds(r * (NSUB * RBLK) + sid * RBLK, RBLK)], dsem))
    for c in cps:
        c.wait()
    plsc.subcore_barrier()

    cprecv = pltpu.async_copy(
        cells_sh.at[pl.ds(sid * (NSUB * RBLK), NSUB * RBLK)], recvbuf, dsem)
    pltpu.sync_copy(hist_sh, histall)
    b0 = 2 * r_own
    acc = [jnp.zeros((16,), jnp.int32) for _ in range(4)]
    cnt0 = []
    cnt1 = []
    for t in range(NSUB):
        c0 = jnp.int32(0)
        c1 = jnp.int32(0)
        for k in range(4):
            hv = histall[pl.ds(t * 64 + 16 * k, 16)]
            acc[k] = acc[k] + hv
            binv = iota + 16 * k
            c0 = c0 + jnp.sum(jnp.where(binv == b0, hv, 0))
            c1 = c1 + jnp.sum(jnp.where(binv == b0 + 1, hv, 0))
        cnt0.append(jnp.minimum(c0, CELL))
        cnt1.append(jnp.minimum(c1, CELL))
    P0 = jnp.int32(0)
    P1 = jnp.int32(0)
    n0 = jnp.int32(0)
    for k in range(4):
        binv = iota + 16 * k
        rv = lax.shift_right_logical(binv, 1)
        is0 = (binv & 1) == 0
        below = rv < r_own
        P0 = P0 + jnp.sum(jnp.where(is0 & below, acc[k], 0))
        P1 = P1 + jnp.sum(jnp.where((~is0) & below, acc[k], 0))
        n0 = n0 + jnp.sum(jnp.where(is0, acc[k], 0))
    m = jnp.minimum(n0, N - n0)

    maxv = jnp.full((16,), IMAX, jnp.int32)

    def pad_body(i, _):
        keybuf[pl.ds(i * 16, 16)] = maxv
        return 0

    lax.fori_loop(0, KPAD // 16, pad_body, 0, unroll=4)
    cprecv.wait()

    off = jnp.int32(0)
    for t in range(NSUB):
        for half, cnt in ((0, cnt0[t]), (1, cnt1[t])):
            cbase = t * RBLK + half * CELL
            for j in range(CELL // 16):
                x = recvbuf[pl.ds(cbase + j * 16, 16)]
                msk = (iota + 16 * j) < cnt
                dst = jnp.minimum(off + jnp.minimum(jnp.int32(16 * j), cnt),
                                  jnp.int32(RCAP))
                plsc.store_compressed(keybuf.at[pl.ds(dst, 16)], x, mask=msk)
            off = jnp.minimum(off + cnt, jnp.int32(RCAP))
    c_r = off


    @plsc.parallel_loop(0, 32, unroll=2)
    def _lvl1(z):
        a = _sort16(keybuf[pl.ds(z * 32, 16)])
        b = _sort16(keybuf[pl.ds(z * 32 + 16, 16)])
        rb = lax.rev(b, (0,))
        keybuf[pl.ds(z * 32, 16)] = _sort16(jnp.minimum(a, rb))
        keybuf[pl.ds(z * 32 + 16, 16)] = _sort16(jnp.maximum(a, rb))

    for lg in range(2, 5):
        un = 1 << lg

        @plsc.parallel_loop(0, 64 // un)
        def _unit(u, un=un):
            base = u * un
            vs = [keybuf[pl.ds((base + j) * 16, 16)] for j in range(un)]
            res = _merge_unit(vs)
            for j in range(un):
                keybuf[pl.ds((base + j) * 16, 16)] = res[j]

    def _split_pair(base, m, i):
        a1 = keybuf[pl.ds((base + i) * 16, 16)]
        a2 = keybuf[pl.ds((base + m - 1 - i) * 16, 16)]
        b1 = keybuf[pl.ds((base + m + i) * 16, 16)]
        b2 = keybuf[pl.ds((base + 2 * m - 1 - i) * 16, 16)]
        rb1 = lax.rev(b1, (0,))
        rb2 = lax.rev(b2, (0,))
        keybuf[pl.ds((base + i) * 16, 16)] = jnp.minimum(a1, rb2)
        keybuf[pl.ds((base + m + i) * 16, 16)] = jnp.maximum(a1, rb2)
        keybuf[pl.ds((base + m - 1 - i) * 16, 16)] = jnp.minimum(a2, rb1)
        keybuf[pl.ds((base + 2 * m - 1 - i) * 16, 16)] = jnp.maximum(a2, rb1)

    def _finish_blocks():
        @plsc.parallel_loop(0, 4)
        def _blk(bi):
            base = bi * 16
            lst = [keybuf[pl.ds((base + j) * 16, 16)] for j in range(16)]
            d = 8
            while d >= 1:
                nxt = list(lst)
                for v in range(16):
                    if (v % (2 * d)) < d:
                        nxt[v] = jnp.minimum(lst[v], lst[v + d])
                        nxt[v + d] = jnp.maximum(lst[v], lst[v + d])
                lst = nxt
                d //= 2
            for j in range(16):
                keybuf[pl.ds((base + j) * 16, 16)] = _sort16(lst[j])

    @plsc.parallel_loop(0, 16, unroll=2)
    def _split5(z):
        _split_pair((z // 8) * 32, 16, z % 8)

    _finish_blocks()

    @plsc.parallel_loop(0, 16, unroll=2)
    def _split6(i):
        _split_pair(0, 32, i)

    @plsc.parallel_loop(0, 32, unroll=4)
    def _dist16(z):
        v = (z // 16) * 32 + z % 16
        x = keybuf[pl.ds(v * 16, 16)]
        y = keybuf[pl.ds((v + 16) * 16, 16)]
        keybuf[pl.ds(v * 16, 16)] = jnp.minimum(x, y)
        keybuf[pl.ds((v + 16) * 16, 16)] = jnp.maximum(x, y)

    _finish_blocks()

    def walk_body(k, carry):
        c1run, accv = carry
        q = keybuf[pl.ds(k * 16, 16)]
        pos = iota + k * 16
        validm = pos < c_r
        f = jnp.where(validm, q & 1, 0)
        incl = plsc.cumsum(f)
        l1 = incl - f + c1run
        l0 = pos - l1
        i0g = P0 + l0
        i1g = P1 + l1
        isg1 = (q & 1) == 1
        iown = jnp.where(isg1, i1g, i0g)
        roth = jnp.where(isg1, i0g, i1g)
        cond = validm & (iown < m)
        val = plsc.bitcast(lax.shift_right_logical(q, 1), jnp.float32)
        sgn = jnp.where(roth > iown, jnp.float32(1.0), jnp.float32(-1.0))
        accv = accv + jnp.where(cond, sgn * val, jnp.float32(0.0))
        return c1run + jnp.sum(f), accv

    _, accv = lax.fori_loop(0, RCAP // 16, walk_body,
                            (jnp.int32(0), jnp.zeros((16,), jnp.float32)),
                            unroll=2)

    sumv[...] = accv
    pltpu.sync_copy(sumv, sums_sh.at[pl.ds(sid * 16, 16)])
    plsc.subcore_barrier()

    @pl.when(sid == 0)
    def _():
        pltpu.sync_copy(sums_sh, sumall)
        tot = jnp.zeros((16,), jnp.float32)
        for t in range(NSUB):
            tot = tot + sumall[pl.ds(t * 16, 16)]
        stot = jnp.sum(tot)
        outv = jnp.where(iota == 0, stot,
                         jnp.where(iota == 1, m.astype(jnp.float32),
                                   jnp.float32(0.0)))
        sumv[...] = outv
        pltpu.sync_copy(sumv, out_hbm.at[pl.ds(cid * 16, 16)])


@functools.partial(jax.jit, static_argnames=())
def _sc_call(pred, grp):
    f = pl.kernel(
        _sc_body,
        out_type=jax.ShapeDtypeStruct((32,), jnp.float32),
        mesh=plsc.VectorSubcoreMesh(core_axis_name="c",
                                    subcore_axis_name="s",
                                    num_cores=2, num_subcores=NSUB),
        scratch_types=[
            pltpu.VMEM((SLICE,), jnp.float32),
            pltpu.VMEM((SLICE,), jnp.int32),
            pltpu.VMEM((64,), jnp.int32),
            pltpu.VMEM((NR * CELL,), jnp.int32),
            pltpu.VMEM((NSUB * 64,), jnp.int32),
            pltpu.VMEM((NSUB * RBLK,), jnp.int32),
            pltpu.VMEM((KPAD,), jnp.int32),
            pltpu.VMEM((16,), jnp.float32),
            pltpu.VMEM((NSUB * 16,), jnp.float32),
            pltpu.VMEM_SHARED((NSUB * 64,), jnp.int32),
            pltpu.VMEM_SHARED((NSUB * NSUB * RBLK,), jnp.int32),
            pltpu.VMEM_SHARED((NSUB * 16,), jnp.float32),
            pltpu.SemaphoreType.DMA,
        ],
        compiler_params=pltpu.CompilerParams(needs_layout_passes=False),
    )
    return f(pred, grp)


def kernel(batch_pred, batch_group):
    out = _sc_call(batch_pred, batch_group.astype(jnp.int32))
    return (out[0] + out[16]) / out[1]
